# Initial kernel scaffold; baseline (speedup 1.0000x reference)
import jax
import jax.numpy as jnp
from jax.experimental import pallas as pl


def kernel(x, edge_index, edge_weight, W1, b1, W2, b2):
    raise NotImplementedError("write your pallas kernel here")


# speedup vs baseline: 3.8884x; 3.8884x over previous
import functools

import jax
import jax.numpy as jnp
from jax import lax
from jax.experimental import pallas as pl
from jax.experimental.pallas import tpu as pltpu
from jax.experimental.pallas import tpu_sc as plsc

NUM_CORES = 2
NUM_SUBCORES = 16
NUM_TILES = NUM_CORES * NUM_SUBCORES
CHUNK = 80
ROW_BLK = 1000


def _spmm_sc(feat, src, dst, w):
    n, d = feat.shape
    e = src.shape[0]
    epw = e // NUM_TILES
    n_chunks = epw // CHUNK
    stripe_stride = (n // NUM_SUBCORES) // 8 * 8
    stripe_width = n - (NUM_SUBCORES - 1) * stripe_stride
    mesh = plsc.VectorSubcoreMesh(
        core_axis_name="c", subcore_axis_name="s",
        num_cores=NUM_CORES, num_subcores=NUM_SUBCORES)

    @functools.partial(
        pl.kernel,
        out_type=jax.ShapeDtypeStruct((NUM_CORES, n, d), jnp.float32),
        mesh=mesh,
        scratch_types=[
            pltpu.VMEM((CHUNK,), jnp.int32),
            pltpu.VMEM((CHUNK,), jnp.int32),
            pltpu.VMEM((CHUNK,), jnp.float32),
            pltpu.VMEM((CHUNK, d), jnp.float32),
            pltpu.VMEM_SHARED((n, d), jnp.float32),
            pltpu.SemaphoreType.DMA,
        ],
        compiler_params=pltpu.CompilerParams(use_tc_tiling_on_sc=False),
    )
    def k(feat_hbm, src_hbm, dst_hbm, w_hbm, zeros_hbm, out_hbm,
          sidx_v, didx_v, w_s, rows_v, acc_sh, sem):
        c = lax.axis_index("c")
        s = lax.axis_index("s")
        r0 = s * stripe_stride
        pltpu.sync_copy(zeros_hbm.at[pl.ds(r0, stripe_width)],
                        acc_sh.at[pl.ds(r0, stripe_width)])
        plsc.subcore_barrier()

        wid = s * NUM_CORES + c
        base = wid * epw

        @pl.loop(0, n_chunks)
        def _(ci):
            e0 = base + ci * CHUNK
            pltpu.sync_copy(src_hbm.at[pl.ds(e0, CHUNK)], sidx_v)
            pltpu.sync_copy(dst_hbm.at[pl.ds(e0, CHUNK)], didx_v)
            pltpu.sync_copy(w_hbm.at[pl.ds(e0, CHUNK)], w_s)
            pltpu.async_copy(feat_hbm.at[sidx_v], rows_v, sem).wait()

            @pl.loop(0, CHUNK, step=16)
            def _(j0):
                w16 = w_s[pl.ds(j0, 16)]
                for jj in range(16):
                    wj = w16[jj]
                    for kk in range(d // 16):
                        sl = (j0 + jj, pl.ds(kk * 16, 16))
                        rows_v[sl] = rows_v[sl] * wj

            pltpu.sync_copy(rows_v, acc_sh.at[didx_v], add=True)

        plsc.subcore_barrier()
        pltpu.sync_copy(acc_sh.at[pl.ds(r0, stripe_width)],
                        out_hbm.at[c, pl.ds(r0, stripe_width)])

    return k(feat, src, dst, w, jnp.zeros((n, d), jnp.float32))


def _matmul_tc(x, w):
    n, kdim = x.shape
    m = w.shape[1]

    def body(x_ref, w_ref, o_ref):
        o_ref[...] = jnp.dot(x_ref[...], w_ref[...],
                             preferred_element_type=jnp.float32)

    return pl.pallas_call(
        body,
        grid=(n // ROW_BLK,),
        in_specs=[
            pl.BlockSpec((ROW_BLK, kdim), lambda i: (i, 0)),
            pl.BlockSpec((kdim, m), lambda i: (0, 0)),
        ],
        out_specs=pl.BlockSpec((ROW_BLK, m), lambda i: (i, 0)),
        out_shape=jax.ShapeDtypeStruct((n, m), jnp.float32),
    )(x, w)


def _combine_relu_matmul_tc(p, b, w):
    _, n, kdim = p.shape
    m = w.shape[1]

    def body(p_ref, b_ref, w_ref, o_ref):
        h = jnp.maximum(p_ref[0] + p_ref[1] + b_ref[...], 0.0)
        o_ref[...] = jnp.dot(h, w_ref[...],
                             preferred_element_type=jnp.float32)

    return pl.pallas_call(
        body,
        grid=(n // ROW_BLK,),
        in_specs=[
            pl.BlockSpec((2, ROW_BLK, kdim), lambda i: (0, i, 0)),
            pl.BlockSpec((1, kdim), lambda i: (0, 0)),
            pl.BlockSpec((kdim, m), lambda i: (0, 0)),
        ],
        out_specs=pl.BlockSpec((ROW_BLK, m), lambda i: (i, 0)),
        out_shape=jax.ShapeDtypeStruct((n, m), jnp.float32),
    )(p, b.reshape(1, kdim), w)


def _combine_logsoftmax_tc(p, b):
    _, n, m = p.shape

    def body(p_ref, b_ref, o_ref):
        y = p_ref[0] + p_ref[1] + b_ref[...]
        z = y - jnp.max(y, axis=1, keepdims=True)
        o_ref[...] = z - jnp.log(jnp.sum(jnp.exp(z), axis=1, keepdims=True))

    return pl.pallas_call(
        body,
        grid=(n // ROW_BLK,),
        in_specs=[
            pl.BlockSpec((2, ROW_BLK, m), lambda i: (0, i, 0)),
            pl.BlockSpec((1, m), lambda i: (0, 0)),
        ],
        out_specs=pl.BlockSpec((ROW_BLK, m), lambda i: (i, 0)),
        out_shape=jax.ShapeDtypeStruct((n, m), jnp.float32),
    )(p, b.reshape(1, m))


def kernel(x, edge_index, edge_weight, W1, b1, W2, b2):
    src = edge_index[0]
    dst = edge_index[1]
    xw1 = _matmul_tc(x, W1)
    p1 = _spmm_sc(xw1, src, dst, edge_weight)
    hw2 = _combine_relu_matmul_tc(p1, b1, W2)
    p2 = _spmm_sc(hw2, src, dst, edge_weight)
    return _combine_logsoftmax_tc(p2, b2)
